# baseline (speedup 1.0000x reference)
import jax
import jax.numpy as jnp
from jax.experimental import pallas as pl

NUM_EMBS = 8192
EMB_DIM = 32
BETA = 0.25
N_TOKENS = 8192
TILE = 1024
GRID = N_TOKENS // TILE
NCHUNK = 16
CW = NUM_EMBS // NCHUNK


def _body(x_ref, sx_ref, sw_ref, wresh_ref, wneg2_ref,
          idx_ref, zq_ref, hist_ref, loss_ref, perp_ref):
    step = pl.program_id(0)

    @pl.when(step == 0)
    def _init():
        hist_ref[...] = jnp.zeros_like(hist_ref)
        loss_ref[...] = jnp.zeros_like(loss_ref)
        perp_ref[...] = jnp.zeros_like(perp_ref)

    x = x_ref[...]

    wneg2 = wneg2_ref[...]
    sw = sw_ref[...]
    sx = sx_ref[...]
    acc_c = jnp.zeros((TILE, CW), jnp.int32)
    acc_v = None
    for c in range(NCHUNK):
        mm2c = jax.lax.dot_general(x, wneg2[c * CW:(c + 1) * CW, :],
                                   (((1,), (1,)), ((), ())),
                                   preferred_element_type=jnp.float32)
        dc = (sx + sw[:, c * CW:(c + 1) * CW]) + mm2c
        if c == 0:
            acc_v = dc
        else:
            m = dc < acc_v
            acc_v = jnp.where(m, dc, acc_v)
            acc_c = jnp.where(m, c, acc_c)
    gmin = jnp.min(acc_v, axis=1, keepdims=True)
    scol = jax.lax.broadcasted_iota(jnp.int32, (TILE, CW), 1)
    jfull = acc_c * CW + scol
    cand = jnp.where(acc_v == gmin, jfull, NUM_EMBS)
    idx = jnp.min(cand, axis=1, keepdims=True)
    idx_ref[...] = idx

    c_star = idx >> 9
    s_star = idx & (CW - 1)
    crow = jax.lax.broadcasted_iota(jnp.int32, (TILE, NCHUNK), 1)
    oh1 = (crow == c_star).astype(jnp.float32)
    oh2 = (scol == s_star).astype(jnp.float32)
    g = jax.lax.dot_general(oh2, wresh_ref[...], (((1,), (0,)), ((), ())),
                            preferred_element_type=jnp.float32)
    q = jnp.zeros((TILE, EMB_DIM), jnp.float32)
    for c in range(NCHUNK):
        q = q + oh1[:, c:c + 1] * g[:, c * EMB_DIM:(c + 1) * EMB_DIM]

    hist_ref[...] += jax.lax.dot_general(
        oh1, oh2, (((0,), (0,)), ((), ())),
        preferred_element_type=jnp.float32)
    zq = x + (q - x)
    zq_ref[...] = zq.T[None]
    loss_ref[...] += jnp.sum((q - x) ** 2)

    @pl.when(step == GRID - 1)
    def _fini():
        loss_ref[...] = (1.0 + BETA) * loss_ref[...] / (N_TOKENS * EMB_DIM)
        probs = hist_ref[...] / N_TOKENS
        ent = -jnp.sum(probs * jnp.log(probs + 1e-10))
        perp_ref[...] = jnp.exp(ent) * jnp.ones_like(perp_ref)


def kernel(z_e_x, W):
    B, C, H, Wd = z_e_x.shape
    x_flat = jnp.transpose(z_e_x, (0, 2, 3, 1)).reshape(-1, EMB_DIM)
    sx = jnp.sum(x_flat ** 2, axis=1, keepdims=True)
    sw = jnp.sum(W ** 2, axis=1)[None, :]
    wresh = jnp.transpose(W.reshape(NCHUNK, CW, EMB_DIM), (1, 0, 2))\
        .reshape(CW, NCHUNK * EMB_DIM)

    idx, zq, hist, loss, perp = pl.pallas_call(
        _body,
        grid=(GRID,),
        in_specs=[
            pl.BlockSpec((TILE, EMB_DIM), lambda i: (i, 0)),
            pl.BlockSpec((TILE, 1), lambda i: (i, 0)),
            pl.BlockSpec((1, NUM_EMBS), lambda i: (0, 0)),
            pl.BlockSpec((CW, NCHUNK * EMB_DIM), lambda i: (0, 0)),
            pl.BlockSpec((NUM_EMBS, EMB_DIM), lambda i: (0, 0)),
        ],
        out_specs=[
            pl.BlockSpec((TILE, 1), lambda i: (i, 0)),
            pl.BlockSpec((1, EMB_DIM, TILE), lambda i: (i, 0, 0)),
            pl.BlockSpec((NCHUNK, CW), lambda i: (0, 0)),
            pl.BlockSpec((1, 1), lambda i: (0, 0)),
            pl.BlockSpec((1, 1), lambda i: (0, 0)),
        ],
        out_shape=[
            jax.ShapeDtypeStruct((N_TOKENS, 1), jnp.int32),
            jax.ShapeDtypeStruct((B, C, H * Wd), jnp.float32),
            jax.ShapeDtypeStruct((NCHUNK, CW), jnp.float32),
            jax.ShapeDtypeStruct((1, 1), jnp.float32),
            jax.ShapeDtypeStruct((1, 1), jnp.float32),
        ],
    )(x_flat, sx, sw, wresh, -2.0 * W)

    return (loss[0, 0], zq.reshape(B, C, H, Wd), perp[0, 0], idx)


# speedup vs baseline: 1.0884x; 1.0884x over previous
import jax
import jax.numpy as jnp
from jax.experimental import pallas as pl

NUM_EMBS = 8192
EMB_DIM = 32
BETA = 0.25
N_TOKENS = 8192
TILE = 1024
GRID = N_TOKENS // TILE
NCHUNK = 16
CW = NUM_EMBS // NCHUNK


def _body(x_ref, sw_ref, wresh_ref, wneg2_ref,
          idx_ref, zq_ref, hist_ref, loss_ref, perp_ref):
    step = pl.program_id(0)

    @pl.when(step == 0)
    def _init():
        hist_ref[...] = jnp.zeros_like(hist_ref)
        loss_ref[...] = jnp.zeros_like(loss_ref)
        perp_ref[...] = jnp.zeros_like(perp_ref)

    xt = x_ref[0]
    x = xt.T
    sx = jnp.sum(x * x, axis=1, keepdims=True)

    wneg2 = wneg2_ref[...]
    sw = sw_ref[...]
    acc_c = jnp.zeros((TILE, CW), jnp.int32)
    acc_v = None
    for c in range(NCHUNK):
        mm2c = jax.lax.dot_general(x, wneg2[c * CW:(c + 1) * CW, :],
                                   (((1,), (1,)), ((), ())),
                                   preferred_element_type=jnp.float32)
        dc = (sx + sw[:, c * CW:(c + 1) * CW]) + mm2c
        if c == 0:
            acc_v = dc
        else:
            m = dc < acc_v
            acc_v = jnp.where(m, dc, acc_v)
            acc_c = jnp.where(m, c, acc_c)
    gmin = jnp.min(acc_v, axis=1, keepdims=True)
    scol = jax.lax.broadcasted_iota(jnp.int32, (TILE, CW), 1)
    jfull = acc_c * CW + scol
    cand = jnp.where(acc_v == gmin, jfull, NUM_EMBS)
    idx = jnp.min(cand, axis=1, keepdims=True)
    idx_ref[...] = idx

    c_star = idx >> 9
    s_star = idx & (CW - 1)
    crow = jax.lax.broadcasted_iota(jnp.int32, (TILE, NCHUNK), 1)
    oh1 = (crow == c_star).astype(jnp.float32)
    oh2 = (scol == s_star).astype(jnp.float32)
    g = jax.lax.dot_general(oh2, wresh_ref[...], (((1,), (0,)), ((), ())),
                            preferred_element_type=jnp.float32)
    q = jnp.zeros((TILE, EMB_DIM), jnp.float32)
    for c in range(NCHUNK):
        q = q + oh1[:, c:c + 1] * g[:, c * EMB_DIM:(c + 1) * EMB_DIM]

    hist_ref[...] += jax.lax.dot_general(
        oh1, oh2, (((0,), (0,)), ((), ())),
        preferred_element_type=jnp.float32)
    zq = x + (q - x)
    zq_ref[...] = zq.T[None]
    loss_ref[...] += jnp.sum((q - x) ** 2)

    @pl.when(step == GRID - 1)
    def _fini():
        loss_ref[...] = (1.0 + BETA) * loss_ref[...] / (N_TOKENS * EMB_DIM)
        probs = hist_ref[...] / N_TOKENS
        ent = -jnp.sum(probs * jnp.log(probs + 1e-10))
        perp_ref[...] = jnp.exp(ent) * jnp.ones_like(perp_ref)


def kernel(z_e_x, W):
    B, C, H, Wd = z_e_x.shape
    z3 = z_e_x.reshape(B, C, H * Wd)
    sw = jnp.sum(W ** 2, axis=1)[None, :]
    wresh = jnp.transpose(W.reshape(NCHUNK, CW, EMB_DIM), (1, 0, 2))\
        .reshape(CW, NCHUNK * EMB_DIM)

    idx, zq, hist, loss, perp = pl.pallas_call(
        _body,
        grid=(GRID,),
        in_specs=[
            pl.BlockSpec((1, EMB_DIM, TILE), lambda i: (i, 0, 0)),
            pl.BlockSpec((1, NUM_EMBS), lambda i: (0, 0)),
            pl.BlockSpec((CW, NCHUNK * EMB_DIM), lambda i: (0, 0)),
            pl.BlockSpec((NUM_EMBS, EMB_DIM), lambda i: (0, 0)),
        ],
        out_specs=[
            pl.BlockSpec((TILE, 1), lambda i: (i, 0)),
            pl.BlockSpec((1, EMB_DIM, TILE), lambda i: (i, 0, 0)),
            pl.BlockSpec((NCHUNK, CW), lambda i: (0, 0)),
            pl.BlockSpec((1, 1), lambda i: (0, 0)),
            pl.BlockSpec((1, 1), lambda i: (0, 0)),
        ],
        out_shape=[
            jax.ShapeDtypeStruct((N_TOKENS, 1), jnp.int32),
            jax.ShapeDtypeStruct((B, C, H * Wd), jnp.float32),
            jax.ShapeDtypeStruct((NCHUNK, CW), jnp.float32),
            jax.ShapeDtypeStruct((1, 1), jnp.float32),
            jax.ShapeDtypeStruct((1, 1), jnp.float32),
        ],
    )(z3, sw, wresh, -2.0 * W)

    return (loss[0, 0], zq.reshape(B, C, H, Wd), perp[0, 0], idx)


# speedup vs baseline: 1.1102x; 1.0200x over previous
import jax
import jax.numpy as jnp
from jax.experimental import pallas as pl
from jax.experimental.pallas import tpu as pltpu

NUM_EMBS = 8192
EMB_DIM = 32
BETA = 0.25
N_TOKENS = 8192
TILE = 1024
GRID = N_TOKENS // TILE
NCHUNK = 16
CW = NUM_EMBS // NCHUNK


def _body(x_ref, w_ref,
          idx_ref, zq_ref, hist_ref, loss_ref, perp_ref,
          sw_ref, wneg2_ref, wresh_ref):
    step = pl.program_id(0)

    @pl.when(step == 0)
    def _init():
        hist_ref[...] = jnp.zeros_like(hist_ref)
        loss_ref[...] = jnp.zeros_like(loss_ref)
        perp_ref[...] = jnp.zeros_like(perp_ref)
        w0 = w_ref[...]
        wneg2_ref[...] = -2.0 * w0
        sw_ref[...] = jnp.sum(w0 * w0, axis=1, keepdims=True).T
        for cc in range(NCHUNK):
            wresh_ref[:, cc * EMB_DIM:(cc + 1) * EMB_DIM] = \
                w0[cc * CW:(cc + 1) * CW, :]

    xt = x_ref[0]
    x = xt.T
    sx = jnp.sum(x * x, axis=1, keepdims=True)

    wneg2 = wneg2_ref[...]
    sw = sw_ref[...]
    acc_c = jnp.zeros((TILE, CW), jnp.int32)
    acc_v = None
    for c in range(NCHUNK):
        mm2c = jax.lax.dot_general(x, wneg2[c * CW:(c + 1) * CW, :],
                                   (((1,), (1,)), ((), ())),
                                   preferred_element_type=jnp.float32)
        dc = (sx + sw[:, c * CW:(c + 1) * CW]) + mm2c
        if c == 0:
            acc_v = dc
        else:
            m = dc < acc_v
            acc_v = jnp.where(m, dc, acc_v)
            acc_c = jnp.where(m, c, acc_c)
    gmin = jnp.min(acc_v, axis=1, keepdims=True)
    scol = jax.lax.broadcasted_iota(jnp.int32, (TILE, CW), 1)
    jfull = acc_c * CW + scol
    cand = jnp.where(acc_v == gmin, jfull, NUM_EMBS)
    idx = jnp.min(cand, axis=1, keepdims=True)
    idx_ref[...] = idx

    c_star = idx >> 9
    s_star = idx & (CW - 1)
    crow = jax.lax.broadcasted_iota(jnp.int32, (TILE, NCHUNK), 1)
    oh1 = (crow == c_star).astype(jnp.float32)
    oh2 = (scol == s_star).astype(jnp.float32)
    g = jax.lax.dot_general(oh2, wresh_ref[...], (((1,), (0,)), ((), ())),
                            preferred_element_type=jnp.float32)
    q = jnp.zeros((TILE, EMB_DIM), jnp.float32)
    for c in range(NCHUNK):
        q = q + oh1[:, c:c + 1] * g[:, c * EMB_DIM:(c + 1) * EMB_DIM]

    hist_ref[...] += jax.lax.dot_general(
        oh1, oh2, (((0,), (0,)), ((), ())),
        preferred_element_type=jnp.float32)
    zq = x + (q - x)
    zq_ref[...] = zq.T[None]
    loss_ref[...] += jnp.sum((q - x) ** 2)

    @pl.when(step == GRID - 1)
    def _fini():
        loss_ref[...] = (1.0 + BETA) * loss_ref[...] / (N_TOKENS * EMB_DIM)
        probs = hist_ref[...] / N_TOKENS
        ent = -jnp.sum(probs * jnp.log(probs + 1e-10))
        perp_ref[...] = jnp.exp(ent) * jnp.ones_like(perp_ref)


def kernel(z_e_x, W):
    B, C, H, Wd = z_e_x.shape
    z3 = z_e_x.reshape(B, C, H * Wd)

    idx, zq, hist, loss, perp = pl.pallas_call(
        _body,
        grid=(GRID,),
        in_specs=[
            pl.BlockSpec((1, EMB_DIM, TILE), lambda i: (i, 0, 0)),
            pl.BlockSpec((NUM_EMBS, EMB_DIM), lambda i: (0, 0)),
        ],
        scratch_shapes=[
            pltpu.VMEM((1, NUM_EMBS), jnp.float32),
            pltpu.VMEM((NUM_EMBS, EMB_DIM), jnp.float32),
            pltpu.VMEM((CW, NCHUNK * EMB_DIM), jnp.float32),
        ],
        out_specs=[
            pl.BlockSpec((TILE, 1), lambda i: (i, 0)),
            pl.BlockSpec((1, EMB_DIM, TILE), lambda i: (i, 0, 0)),
            pl.BlockSpec((NCHUNK, CW), lambda i: (0, 0)),
            pl.BlockSpec((1, 1), lambda i: (0, 0)),
            pl.BlockSpec((1, 1), lambda i: (0, 0)),
        ],
        out_shape=[
            jax.ShapeDtypeStruct((N_TOKENS, 1), jnp.int32),
            jax.ShapeDtypeStruct((B, C, H * Wd), jnp.float32),
            jax.ShapeDtypeStruct((NCHUNK, CW), jnp.float32),
            jax.ShapeDtypeStruct((1, 1), jnp.float32),
            jax.ShapeDtypeStruct((1, 1), jnp.float32),
        ],
    )(z3, W)

    return (loss[0, 0], zq.reshape(B, C, H, Wd), perp[0, 0], idx)
